# baseline (speedup 1.0000x reference)
import functools
import math

import jax
import jax.numpy as jnp
from jax.experimental import pallas as pl
from jax.experimental.pallas import tpu as pltpu


def _fake_quant(x):
    s = jnp.maximum(jnp.max(jnp.abs(x), axis=1, keepdims=True) / 7.0, 1e-8)
    return jnp.clip(jnp.round(x * (1.0 / s)), -8.0, 7.0) * s


def _attn_body(q_ref, k_hbm, v_hbm, o_ref,
               kbuf, vbuf, sem,
               *, S, Qn, B, KVH, G, D):
    i = pl.program_id(0)
    n = pl.num_programs(0)
    b, h = i // KVH, i % KVH

    H2 = S // 2

    def k_copy(slot, bb, hh, half):
        sl = pl.ds(half * H2, H2)
        return pltpu.make_async_copy(
            k_hbm.at[bb, sl, hh, :], kbuf.at[slot, sl], sem.at[slot, half])

    def v_copy(slot, bb, hh, half):
        sl = pl.ds(half * H2, H2)
        return pltpu.make_async_copy(
            v_hbm.at[bb, sl, hh, :], vbuf.at[slot, sl], sem.at[slot, 2 + half])

    def start_all(slot, bb, hh):
        k_copy(slot, bb, hh, 0).start()
        k_copy(slot, bb, hh, 1).start()
        v_copy(slot, bb, hh, 0).start()
        v_copy(slot, bb, hh, 1).start()

    slot = i % 2

    @pl.when(i == 0)
    def _prologue():
        start_all(0, b, h)

    @pl.when(i + 1 < n)
    def _prefetch():
        start_all(1 - slot, (i + 1) // KVH, (i + 1) % KVH)

    k_copy(slot, b, h, 0).wait()
    k_copy(slot, b, h, 1).wait()
    km = kbuf[slot]
    kq = _fake_quant(km)

    v_copy(slot, b, h, 0).wait()
    v_copy(slot, b, h, 1).wait()
    vm = vbuf[slot]
    vq = _fake_quant(vm)

    qm = q_ref[0, 0]
    scores = jax.lax.dot_general(
        qm, kq, (((1,), (1,)), ((), ())),
        preferred_element_type=jnp.float32) * (1.0 / math.sqrt(D))

    rows = jax.lax.broadcasted_iota(jnp.int32, (Qn * G, S), 0)
    cols = jax.lax.broadcasted_iota(jnp.int32, (Qn * G, S), 1)
    qpos = (S - Qn) + rows // G
    scores = jnp.where(cols <= qpos, scores, jnp.float32(-1e30))

    m = jnp.max(scores, axis=1, keepdims=True)
    p = jnp.exp(scores - m)
    l = jnp.sum(p, axis=1, keepdims=True)
    o = jax.lax.dot_general(
        p, vq, (((1,), (0,)), ((), ())),
        preferred_element_type=jnp.float32)
    o_ref[0, 0] = o / l


def kernel(q, k, v, block_table):
    B, Qn, H, D = q.shape
    _, S, KVH, _ = k.shape
    G = H // KVH

    qg = (q.reshape(B, Qn, KVH, G, D)
           .transpose(0, 2, 1, 3, 4)
           .reshape(B, KVH, Qn * G, D))

    out = pl.pallas_call(
        functools.partial(_attn_body, S=S, Qn=Qn, B=B, KVH=KVH, G=G, D=D),
        grid=(B * KVH,),
        in_specs=[
            pl.BlockSpec((1, 1, Qn * G, D),
                         lambda i: (i // KVH, i % KVH, 0, 0)),
            pl.BlockSpec(memory_space=pl.ANY),
            pl.BlockSpec(memory_space=pl.ANY),
        ],
        out_specs=pl.BlockSpec((1, 1, Qn * G, D),
                               lambda i: (i // KVH, i % KVH, 0, 0)),
        out_shape=jax.ShapeDtypeStruct((B, KVH, Qn * G, D), jnp.float32),
        scratch_shapes=[
            pltpu.VMEM((2, S, D), jnp.float32),
            pltpu.VMEM((2, S, D), jnp.float32),
            pltpu.SemaphoreType.DMA((2, 4)),
        ],
        compiler_params=pltpu.CompilerParams(
            dimension_semantics=("arbitrary",),
        ),
    )(qg, k, v)

    return (out.reshape(B, KVH, Qn, G, D)
               .transpose(0, 2, 1, 3, 4)
               .reshape(B, Qn, H, D))
